# Initial kernel scaffold; baseline (speedup 1.0000x reference)
import jax
import jax.numpy as jnp
from jax.experimental import pallas as pl


def kernel(x, label_feats, W1, b1, W2, b2, Wl1, bl1, Wl2, bl2, ln1_g, ln1_b, ln2_g, ln2_b, lnl1_g, lnl1_b, lnl2_g, lnl2_b, a1, a2, al1, al2, Wq, Wk, Wv, gamma, Wg, bg, We, be):
    raise NotImplementedError("write your pallas kernel here")


# speedup vs baseline: 16.5820x; 16.5820x over previous
import jax
import jax.numpy as jnp
from jax.experimental import pallas as pl
from jax.experimental.pallas import tpu as pltpu

_B, _NF, _NLF, _NFEAT, _NCLASS, _HID, _E, _TOPK = 1024, 6, 3, 256, 349, 256, 8, 2
_C = _NF + _NLF
_TB = 256


def _sigma_kernel(wqt_ref, wkt_ref, wvt_ref, out_ref):
    def sig(wt):
        g0 = jax.lax.dot_general(wt, wt, (((0,), (0,)), ((), ())),
                                 preferred_element_type=jnp.float32)
        def body(_, g):
            g = jnp.dot(g, g, preferred_element_type=jnp.float32)
            return g * (1.0 / jnp.max(jnp.abs(g)))
        g = jax.lax.fori_loop(0, 8, body, g0 * (1.0 / jnp.max(jnp.abs(g0))))
        v = jnp.sum(g, axis=1, keepdims=True)
        gv = jnp.dot(g0, v, preferred_element_type=jnp.float32)
        lam = jnp.sum(v * gv) / jnp.sum(v * v)
        return jnp.sqrt(lam)

    sq = sig(wqt_ref[...])
    sk = sig(wkt_ref[...])
    sv = sig(wvt_ref[...])
    out_ref[0, 0] = 1.0 / (sq * sk)
    out_ref[0, 1] = 1.0 / sv


def _main_kernel(scal_ref, x_ref, lf_ref, W1_ref, b1_ref, W2_ref, b2_ref,
                 Wl1_ref, bl1_ref, Wl2_ref, bl2_ref,
                 g1_ref, be1_ref, g2_ref, be2_ref,
                 gl1_ref, bel1_ref, gl2_ref, bel2_ref,
                 WqT_ref, WkT_ref, WvT_ref, Wg_ref, bg_ref, We_ref, bee_ref,
                 out_ref):
    a1 = scal_ref[0, 0]
    a2 = scal_ref[0, 1]
    al1 = scal_ref[0, 2]
    al2 = scal_ref[0, 3]
    gamma = scal_ref[0, 4]
    inv_qk = scal_ref[0, 5]
    inv_v = scal_ref[0, 6]

    def proj_layer(zs, W_ref, b_ref, g_ref, beta_ref, a, nc):
        hs = [jnp.dot(zs[c], W_ref[c], preferred_element_type=jnp.float32)
              + b_ref[c:c + 1, :] for c in range(nc)]
        n = nc * _HID
        tot = sum(jnp.sum(h, axis=1, keepdims=True) for h in hs)
        mean = tot * (1.0 / n)
        tot2 = sum(jnp.sum(h * h, axis=1, keepdims=True) for h in hs)
        var = tot2 * (1.0 / n) - mean * mean
        inv = jax.lax.rsqrt(var + 1e-5)
        outs = []
        for c in range(nc):
            o = (hs[c] - mean) * inv * g_ref[c:c + 1, :] + beta_ref[c:c + 1, :]
            outs.append(jnp.where(o > 0, o, a * o))
        return outs

    xs = [x_ref[:, c, :] for c in range(_NF)]
    hs = proj_layer(xs, W1_ref, b1_ref, g1_ref, be1_ref, a1, _NF)
    hs = proj_layer(hs, W2_ref, b2_ref, g2_ref, be2_ref, a2, _NF)
    lfs = [lf_ref[:, c, :] for c in range(_NLF)]
    hls = proj_layer(lfs, Wl1_ref, bl1_ref, gl1_ref, bel1_ref, al1, _NLF)
    hls = proj_layer(hls, Wl2_ref, bl2_ref, gl2_ref, bel2_ref, al2, _NLF)
    zs = hs + hls

    Fs = [jnp.dot(z, WqT_ref[...], preferred_element_type=jnp.float32) for z in zs]
    Gs = [jnp.dot(z, WkT_ref[...], preferred_element_type=jnp.float32) for z in zs]
    Vs = [jnp.dot(z, WvT_ref[...], preferred_element_type=jnp.float32) for z in zs]

    o_zs = []
    for d in range(_C):
        col = jnp.concatenate(
            [jnp.sum(Fs[c] * Gs[d], axis=1, keepdims=True) for c in range(_C)],
            axis=1) * inv_qk
        col = jnp.maximum(col, 0.0)
        m = jnp.max(col, axis=1, keepdims=True)
        e = jnp.exp(col - m)
        beta_d = e * (1.0 / jnp.sum(e, axis=1, keepdims=True))
        acc = jnp.zeros_like(Vs[0])
        for c in range(_C):
            acc = acc + beta_d[:, c:c + 1] * Vs[c]
        o_zs.append(gamma * inv_v * acc + zs[d])

    logits = bg_ref[0:1, :]
    for c in range(_C):
        logits = logits + jnp.dot(o_zs[c], Wg_ref[c],
                                  preferred_element_type=jnp.float32)
    idx = jax.lax.broadcasted_iota(jnp.int32, (logits.shape[0], _E), 1)
    m1 = jnp.max(logits, axis=1, keepdims=True)
    i1 = jnp.min(jnp.where(logits == m1, idx, _E), axis=1, keepdims=True)
    masked = jnp.where(idx == i1, -jnp.inf, logits)
    m2 = jnp.max(masked, axis=1, keepdims=True)
    i2 = jnp.min(jnp.where(masked == m2, idx, _E), axis=1, keepdims=True)
    s1 = 1.0 / (1.0 + jnp.exp(m2 - m1))
    s2 = 1.0 - s1

    acc = jnp.zeros_like(out_ref)
    for e in range(_E):
        ee = bee_ref[e:e + 1, :]
        exp_out = ee + jnp.dot(o_zs[0], We_ref[e, 0],
                               preferred_element_type=jnp.float32)
        for c in range(1, _C):
            exp_out = exp_out + jnp.dot(o_zs[c], We_ref[e, c],
                                        preferred_element_type=jnp.float32)
        w = s1 * (i1 == e).astype(jnp.float32) + s2 * (i2 == e).astype(jnp.float32)
        acc = acc + w * exp_out
    out_ref[...] = acc


def kernel(x, label_feats, W1, b1, W2, b2, Wl1, bl1, Wl2, bl2,
           ln1_g, ln1_b, ln2_g, ln2_b, lnl1_g, lnl1_b, lnl2_g, lnl2_b,
           a1, a2, al1, al2, Wq, Wk, Wv, gamma, Wg, bg, We, be):
    WqT = Wq.T
    WkT = Wk.T
    WvT = Wv.T

    inv = pl.pallas_call(
        _sigma_kernel,
        out_shape=jax.ShapeDtypeStruct((1, 2), jnp.float32),
        out_specs=pl.BlockSpec(memory_space=pltpu.SMEM),
    )(WqT, WkT, WvT)

    scal = jnp.concatenate([a1, a2, al1, al2, gamma, inv[0]]).reshape(1, 7)
    Wg_r = Wg.reshape(_C, _HID, _E)
    bg_r = bg.reshape(1, _E)
    We_r = We.reshape(_E, _C, _HID, _HID)

    nblk = _B // _TB
    full = lambda arr: pl.BlockSpec(arr.shape, lambda i: (0,) * arr.ndim)

    out = pl.pallas_call(
        _main_kernel,
        grid=(nblk,),
        in_specs=[
            pl.BlockSpec(memory_space=pltpu.SMEM),
            pl.BlockSpec((_TB, _NF, _NFEAT), lambda i: (i, 0, 0)),
            pl.BlockSpec((_TB, _NLF, _NCLASS), lambda i: (i, 0, 0)),
            full(W1), full(b1), full(W2), full(b2),
            full(Wl1), full(bl1), full(Wl2), full(bl2),
            full(ln1_g), full(ln1_b), full(ln2_g), full(ln2_b),
            full(lnl1_g), full(lnl1_b), full(lnl2_g), full(lnl2_b),
            full(WqT), full(WkT), full(WvT),
            full(Wg_r), full(bg_r), full(We_r), full(be),
        ],
        out_specs=pl.BlockSpec((_TB, _HID), lambda i: (i, 0)),
        out_shape=jax.ShapeDtypeStruct((_B, _HID), jnp.float32),
        compiler_params=pltpu.CompilerParams(
            dimension_semantics=("arbitrary",),
        ),
    )(scal, x, label_feats, W1, b1, W2, b2, Wl1, bl1, Wl2, bl2,
      ln1_g, ln1_b, ln2_g, ln2_b, lnl1_g, lnl1_b, lnl2_g, lnl2_b,
      WqT, WkT, WvT, Wg_r, bg_r, We_r, be)
    return out


# speedup vs baseline: 17.4164x; 1.0503x over previous
import jax
import jax.numpy as jnp
from jax.experimental import pallas as pl
from jax.experimental.pallas import tpu as pltpu

_B, _NF, _NLF, _NFEAT, _NCLASS, _HID, _E, _TOPK = 1024, 6, 3, 256, 349, 256, 8, 2
_C = _NF + _NLF
_TB = 256


def _spectral_inv(w):
    g0 = jax.lax.dot_general(w, w, (((1,), (1,)), ((), ())),
                             preferred_element_type=jnp.float32)

    def body(_, g):
        g = jnp.dot(g, g, preferred_element_type=jnp.float32)
        return g * (1.0 / jnp.max(jnp.abs(g)))

    g = jax.lax.fori_loop(0, 8, body, g0 * (1.0 / jnp.max(jnp.abs(g0))))
    v = jnp.sum(g, axis=1, keepdims=True)
    gv = jnp.dot(g0, v, preferred_element_type=jnp.float32)
    lam = jnp.sum(v * gv) / jnp.sum(v * v)
    return jax.lax.rsqrt(lam)


def _proj_attn_kernel(x_ref, lf_ref, W1_ref, b1_ref, W2_ref, b2_ref,
                      Wl1_ref, bl1_ref, Wl2_ref, bl2_ref,
                      g1_ref, be1_ref, g2_ref, be2_ref,
                      gl1_ref, bel1_ref, gl2_ref, bel2_ref,
                      Wq_ref, Wk_ref, Wv_ref, Wg_ref, bg_ref,
                      a1_ref, a2_ref, al1_ref, al2_ref, gamma_ref,
                      flat_ref, w_ref, inv_scr):
    i = pl.program_id(0)

    @pl.when(i == 0)
    def _():
        inv_scr[0] = _spectral_inv(Wq_ref[...]) * _spectral_inv(Wk_ref[...])
        inv_scr[1] = _spectral_inv(Wv_ref[...])

    inv_qk = inv_scr[0]
    inv_v = inv_scr[1]
    gamma = gamma_ref[0]

    def proj_layer(zs, W_ref, b_ref, g_ref, beta_ref, a, nc):
        hs = [jnp.dot(zs[c], W_ref[c], preferred_element_type=jnp.float32)
              + b_ref[c:c + 1, :] for c in range(nc)]
        n = nc * _HID
        tot = sum(jnp.sum(h, axis=1, keepdims=True) for h in hs)
        mean = tot * (1.0 / n)
        tot2 = sum(jnp.sum(h * h, axis=1, keepdims=True) for h in hs)
        var = tot2 * (1.0 / n) - mean * mean
        inv = jax.lax.rsqrt(var + 1e-5)
        outs = []
        for c in range(nc):
            o = (hs[c] - mean) * inv * g_ref[c:c + 1, :] + beta_ref[c:c + 1, :]
            outs.append(jnp.where(o > 0, o, a * o))
        return outs

    xs = [x_ref[:, c, :] for c in range(_NF)]
    hs = proj_layer(xs, W1_ref, b1_ref, g1_ref, be1_ref, a1_ref[0], _NF)
    hs = proj_layer(hs, W2_ref, b2_ref, g2_ref, be2_ref, a2_ref[0], _NF)
    lfs = [lf_ref[:, c, :] for c in range(_NLF)]
    hls = proj_layer(lfs, Wl1_ref, bl1_ref, gl1_ref, bel1_ref, al1_ref[0], _NLF)
    hls = proj_layer(hls, Wl2_ref, bl2_ref, gl2_ref, bel2_ref, al2_ref[0], _NLF)
    zs = hs + hls

    dims = (((1,), (1,)), ((), ()))
    Fs = [jax.lax.dot_general(z, Wq_ref[...], dims,
                              preferred_element_type=jnp.float32) for z in zs]
    Gs = [jax.lax.dot_general(z, Wk_ref[...], dims,
                              preferred_element_type=jnp.float32) for z in zs]
    Vs = [jax.lax.dot_general(z, Wv_ref[...], dims,
                              preferred_element_type=jnp.float32) for z in zs]

    logits = bg_ref[0:1, :]
    for d in range(_C):
        col = jnp.concatenate(
            [jnp.sum(Fs[c] * Gs[d], axis=1, keepdims=True) for c in range(_C)],
            axis=1) * inv_qk
        col = jnp.maximum(col, 0.0)
        m = jnp.max(col, axis=1, keepdims=True)
        e = jnp.exp(col - m)
        beta_d = e * (1.0 / jnp.sum(e, axis=1, keepdims=True))
        acc = jnp.zeros_like(Vs[0])
        for c in range(_C):
            acc = acc + beta_d[:, c:c + 1] * Vs[c]
        o_z = gamma * inv_v * acc + zs[d]
        flat_ref[:, d * _HID:(d + 1) * _HID] = o_z
        logits = logits + jnp.dot(o_z, Wg_ref[d],
                                  preferred_element_type=jnp.float32)

    idx = jax.lax.broadcasted_iota(jnp.int32, logits.shape, 1)
    m1 = jnp.max(logits, axis=1, keepdims=True)
    i1 = jnp.min(jnp.where(logits == m1, idx, _E), axis=1, keepdims=True)
    masked = jnp.where(idx == i1, -jnp.inf, logits)
    m2 = jnp.max(masked, axis=1, keepdims=True)
    i2 = jnp.min(jnp.where(masked == m2, idx, _E), axis=1, keepdims=True)
    s1 = 1.0 / (1.0 + jnp.exp(m2 - m1))
    s2 = 1.0 - s1
    w_ref[...] = s1 * (idx == i1).astype(jnp.float32) \
        + s2 * (idx == i2).astype(jnp.float32)


def _moe_kernel(flat_ref, w_ref, We_ref, be_ref, out_ref):
    e = pl.program_id(0)

    @pl.when(e == 0)
    def _():
        out_ref[...] = jnp.zeros_like(out_ref)

    idx = jax.lax.broadcasted_iota(jnp.int32, w_ref.shape, 1)
    w_e = jnp.sum(jnp.where(idx == e, w_ref[...], 0.0), axis=1, keepdims=True)
    exp_out = jnp.dot(flat_ref[...], We_ref[0],
                      preferred_element_type=jnp.float32) + be_ref[0]
    out_ref[...] = out_ref[...] + w_e * exp_out


def kernel(x, label_feats, W1, b1, W2, b2, Wl1, bl1, Wl2, bl2,
           ln1_g, ln1_b, ln2_g, ln2_b, lnl1_g, lnl1_b, lnl2_g, lnl2_b,
           a1, a2, al1, al2, Wq, Wk, Wv, gamma, Wg, bg, We, be):
    Wg_r = Wg.reshape(_C, _HID, _E)
    bg_r = bg.reshape(1, _E)

    nblk = _B // _TB
    full = lambda arr: pl.BlockSpec(arr.shape, lambda i: (0,) * arr.ndim)
    smem = pl.BlockSpec(memory_space=pltpu.SMEM)

    flat, w = pl.pallas_call(
        _proj_attn_kernel,
        grid=(nblk,),
        in_specs=[
            pl.BlockSpec((_TB, _NF, _NFEAT), lambda i: (i, 0, 0)),
            pl.BlockSpec((_TB, _NLF, _NCLASS), lambda i: (i, 0, 0)),
            full(W1), full(b1), full(W2), full(b2),
            full(Wl1), full(bl1), full(Wl2), full(bl2),
            full(ln1_g), full(ln1_b), full(ln2_g), full(ln2_b),
            full(lnl1_g), full(lnl1_b), full(lnl2_g), full(lnl2_b),
            full(Wq), full(Wk), full(Wv), full(Wg_r), full(bg_r),
            smem, smem, smem, smem, smem,
        ],
        out_specs=[
            pl.BlockSpec((_TB, _C * _HID), lambda i: (i, 0)),
            pl.BlockSpec((_TB, _E), lambda i: (i, 0)),
        ],
        out_shape=[
            jax.ShapeDtypeStruct((_B, _C * _HID), jnp.float32),
            jax.ShapeDtypeStruct((_B, _E), jnp.float32),
        ],
        scratch_shapes=[pltpu.SMEM((2,), jnp.float32)],
        compiler_params=pltpu.CompilerParams(
            dimension_semantics=("arbitrary",),
        ),
    )(x, label_feats, W1, b1, W2, b2, Wl1, bl1, Wl2, bl2,
      ln1_g, ln1_b, ln2_g, ln2_b, lnl1_g, lnl1_b, lnl2_g, lnl2_b,
      Wq, Wk, Wv, Wg_r, bg_r, a1, a2, al1, al2, gamma)

    out = pl.pallas_call(
        _moe_kernel,
        grid=(_E,),
        in_specs=[
            pl.BlockSpec((_B, _C * _HID), lambda e: (0, 0)),
            pl.BlockSpec((_B, _E), lambda e: (0, 0)),
            pl.BlockSpec((1, _C * _HID, _HID), lambda e: (e, 0, 0)),
            pl.BlockSpec((1, 1, _HID), lambda e: (e, 0, 0)),
        ],
        out_specs=pl.BlockSpec((_B, _HID), lambda e: (0, 0)),
        out_shape=jax.ShapeDtypeStruct((_B, _HID), jnp.float32),
        compiler_params=pltpu.CompilerParams(
            dimension_semantics=("arbitrary",),
        ),
    )(flat, w, We, be.reshape(_E, 1, _HID))
    return out
